# Initial kernel scaffold; baseline (speedup 1.0000x reference)
import jax
import jax.numpy as jnp
from jax.experimental import pallas as pl


def kernel(xyz, planes_xy0, planes_xz0, planes_yz0, planes_xy1, planes_xz1, planes_yz1):
    raise NotImplementedError("write your pallas kernel here")


# speedup vs baseline: 81.6033x; 81.6033x over previous
import functools

import jax
import jax.numpy as jnp
from jax import lax
from jax.experimental import pallas as pl
from jax.experimental.pallas import tpu as pltpu
from jax.experimental.pallas import tpu_sc as plsc

N = 524288
C = 32
NC, NS = 2, 16
NW = NC * NS
NPW = N // NW
P = 64
NCHUNK = NPW // P
G16 = P // 16

_SIZES = (128, 128, 128, 512, 512, 512)
_OFFS = (0, 16384, 32768, 49152, 311296, 573440)
_COLROW = ((0, 1), (0, 2), (1, 2), (0, 1), (0, 2), (1, 2))
_FROW = {(c, s): (0 if s == 128 else 1) * 3 + c for c in range(3) for s in (128, 512)}

_NIDX = 24 * P
_NROWS128 = _NIDX // 128


def _splat(vec, lane):
  return vec[jnp.full((16,), lane, jnp.int32)]


def _body(xs, ys, zs, table, out_hbm, xv, yv, zv, fracs, idx2d, dest, outst, gsem):
  wid = lax.axis_index("s") * NC + lax.axis_index("c")
  base0 = wid * NPW

  @pl.loop(0, NCHUNK)
  def _chunk(k):
    base = base0 + k * P
    pltpu.sync_copy(xs.at[pl.ds(base, P)], xv)
    pltpu.sync_copy(ys.at[pl.ds(base, P)], yv)
    pltpu.sync_copy(zs.at[pl.ds(base, P)], zv)

    @pl.loop(0, G16)
    def _idx(g):
      col = g * 16
      coords = (xv[pl.ds(col, 16)], yv[pl.ds(col, 16)], zv[pl.ds(col, 16)])
      q = {}
      for s in (128, 512):
        for ci, cval in enumerate(coords):
          t = jnp.minimum(jnp.maximum(cval * float(s - 1), 0.0), float(s - 1))
          c0 = t.astype(jnp.int32)
          frac = t - c0.astype(jnp.float32)
          dc = jnp.minimum(c0 + 1, s - 1) - c0
          q[(ci, s)] = (c0, dc)
          fracs[_FROW[(ci, s)], pl.ds(col, 16)] = frac
      for p in range(6):
        s = _SIZES[p]
        cc, rc = _COLROW[p]
        c0, dc = q[(cc, s)]
        r0, dr = q[(rc, s)]
        i00 = (r0 * s + c0) + _OFFS[p]
        i01 = i00 + dc
        i10 = i00 + dr * s
        i11 = i10 + dc
        for corner, iv in enumerate((i00, i01, i10, i11)):
          pos = (p * 4 + corner) * P
          idx2d[pos // 128, pl.ds((pos % 128) + col, 16)] = iv

    cps = [
        pltpu.async_copy(
            table.at[idx2d.at[g]], dest.at[pl.ds(g * 128, 128)], gsem
        )
        for g in range(_NROWS128)
    ]
    for cp in cps:
      cp.wait()

    @pl.loop(0, G16)
    def _fma(g):
      col = g * 16
      fv = [fracs[r, pl.ds(col, 16)] for r in range(6)]
      for j in range(16):
        prel = col + j
        accs = []
        for half in range(2):
          acc_lo = jnp.zeros((16,), jnp.float32)
          acc_hi = jnp.zeros((16,), jnp.float32)
          for p in range(3 * half, 3 * half + 3):
            s = _SIZES[p]
            cc, rc = _COLROW[p]
            wx = _splat(fv[_FROW[(cc, s)]], j)
            wy = _splat(fv[_FROW[(rc, s)]], j)
            r = []
            for corner in range(4):
              row = (p * 4 + corner) * P + prel
              r.append((dest[row, pl.ds(0, 16)], dest[row, pl.ds(16, 16)]))
            a_lo = r[0][0] + wx * (r[1][0] - r[0][0])
            a_hi = r[0][1] + wx * (r[1][1] - r[0][1])
            b_lo = r[2][0] + wx * (r[3][0] - r[2][0])
            b_hi = r[2][1] + wx * (r[3][1] - r[2][1])
            acc_lo = acc_lo + a_lo + wy * (b_lo - a_lo)
            acc_hi = acc_hi + a_hi + wy * (b_hi - a_hi)
          accs.append((acc_lo, acc_hi))
        outst[prel, pl.ds(0, 16)] = accs[0][0]
        outst[prel, pl.ds(16, 16)] = accs[0][1]
        outst[prel, pl.ds(32, 16)] = accs[1][0]
        outst[prel, pl.ds(48, 16)] = accs[1][1]

    pltpu.sync_copy(outst, out_hbm.at[pl.ds(base, P)])


_tri_plane_sc = functools.partial(
    pl.kernel,
    out_type=jax.ShapeDtypeStruct((N, 2 * C), jnp.float32),
    mesh=plsc.VectorSubcoreMesh(
        core_axis_name="c", subcore_axis_name="s", num_cores=NC, num_subcores=NS
    ),
    scratch_types=[
        pltpu.VMEM((P,), jnp.float32),
        pltpu.VMEM((P,), jnp.float32),
        pltpu.VMEM((P,), jnp.float32),
        pltpu.VMEM((6, P), jnp.float32),
        pltpu.VMEM((_NROWS128, 128), jnp.int32),
        pltpu.VMEM((_NIDX, C), jnp.float32),
        pltpu.VMEM((P, 2 * C), jnp.float32),
        pltpu.SemaphoreType.DMA,
    ],
    compiler_params=pltpu.CompilerParams(use_tc_tiling_on_sc=False),
)(_body)


def kernel(xyz, planes_xy0, planes_xz0, planes_yz0, planes_xy1, planes_xz1,
           planes_yz1):
  planes = (planes_xy0, planes_xz0, planes_yz0, planes_xy1, planes_xz1,
            planes_yz1)
  table = jnp.concatenate([p.reshape(C, -1).T for p in planes], axis=0)
  xs = xyz[:, 0]
  ys = xyz[:, 1]
  zs = xyz[:, 2]
  return _tri_plane_sc(xs, ys, zs, table)


# speedup vs baseline: 85.5295x; 1.0481x over previous
import functools

import jax
import jax.numpy as jnp
from jax import lax
from jax.experimental import pallas as pl
from jax.experimental.pallas import tpu as pltpu
from jax.experimental.pallas import tpu_sc as plsc

N = 524288
C = 32
NC, NS = 2, 16
NW = NC * NS
NPW = N // NW
P = 64
NCHUNK = NPW // P
G16 = P // 16

_SIZES = (128, 128, 128, 512, 512, 512)
_OFFS = (0, 16384, 32768, 49152, 311296, 573440)
_COLROW = ((0, 1), (0, 2), (1, 2), (0, 1), (0, 2), (1, 2))
_FROW = {(c, s): (0 if s == 128 else 1) * 3 + c for c in range(3) for s in (128, 512)}

_NIDX = 24 * P
_NROWS128 = _NIDX // 128


def _splat(vec, lane):
  return vec[jnp.full((16,), lane, jnp.int32)]


def _body(xs, ys, zs, table, out_hbm,
          xv0, yv0, zv0, fracs0, idx0, dest0, sem0,
          xv1, yv1, zv1, fracs1, idx1, dest1, sem1,
          outst):
  wid = lax.axis_index("s") * NC + lax.axis_index("c")
  base0 = wid * NPW
  bufs = ((xv0, yv0, zv0, fracs0, idx0, dest0, sem0),
          (xv1, yv1, zv1, fracs1, idx1, dest1, sem1))

  def stage(buf, k):
    xv, yv, zv, fracs, idx2d, dest, sem = buf
    base = base0 + k * P
    pltpu.sync_copy(xs.at[pl.ds(base, P)], xv)
    pltpu.sync_copy(ys.at[pl.ds(base, P)], yv)
    pltpu.sync_copy(zs.at[pl.ds(base, P)], zv)

    @pl.loop(0, G16)
    def _idx(g):
      col = g * 16
      coords = (xv[pl.ds(col, 16)], yv[pl.ds(col, 16)], zv[pl.ds(col, 16)])
      q = {}
      for s in (128, 512):
        for ci, cval in enumerate(coords):
          t = jnp.minimum(jnp.maximum(cval * float(s - 1), 0.0), float(s - 1))
          c0 = t.astype(jnp.int32)
          frac = t - c0.astype(jnp.float32)
          dc = jnp.minimum(c0 + 1, s - 1) - c0
          q[(ci, s)] = (c0, dc)
          fracs[_FROW[(ci, s)], pl.ds(col, 16)] = frac
      for p in range(6):
        s = _SIZES[p]
        cc, rc = _COLROW[p]
        c0, dc = q[(cc, s)]
        r0, dr = q[(rc, s)]
        i00 = (r0 * s + c0) + _OFFS[p]
        i01 = i00 + dc
        i10 = i00 + dr * s
        i11 = i10 + dc
        for corner, iv in enumerate((i00, i01, i10, i11)):
          pos = (p * 4 + corner) * P
          idx2d[pos // 128, pl.ds((pos % 128) + col, 16)] = iv

    for g in range(_NROWS128):
      pltpu.async_copy(table.at[idx2d.at[g]], dest.at[pl.ds(g * 128, 128)], sem)

  def finish(buf, k):
    xv, yv, zv, fracs, idx2d, dest, sem = buf
    base = base0 + k * P
    for g in range(_NROWS128):
      pltpu.make_async_copy(
          table.at[idx2d.at[g]], dest.at[pl.ds(g * 128, 128)], sem
      ).wait()

    @pl.loop(0, G16)
    def _fma(g):
      col = g * 16
      fv = [fracs[r, pl.ds(col, 16)] for r in range(6)]
      for j in range(16):
        prel = col + j
        for half in range(2):
          acc = jnp.zeros((C,), jnp.bfloat16)
          for p in range(3 * half, 3 * half + 3):
            s = _SIZES[p]
            cc, rc = _COLROW[p]
            wx = _splat(fv[_FROW[(cc, s)]], j)
            wy = _splat(fv[_FROW[(rc, s)]], j)
            wxp = plsc.pack(wx, wx, format=plsc.PackFormat.INTERLEAVED)
            wyp = plsc.pack(wy, wy, format=plsc.PackFormat.INTERLEAVED)
            r = [dest[(p * 4 + corner) * P + prel, :] for corner in range(4)]
            a = r[0] + wxp * (r[1] - r[0])
            b = r[2] + wxp * (r[3] - r[2])
            acc = acc + (a + wyp * (b - a))
          lo, hi = plsc.unpack(
              acc, format=plsc.PackFormat.INTERLEAVED,
              preferred_element_type=jnp.float32)
          outst[prel, pl.ds(32 * half, 16)] = lo
          outst[prel, pl.ds(32 * half + 16, 16)] = hi

    pltpu.sync_copy(outst, out_hbm.at[pl.ds(base, P)])

  stage(bufs[0], 0)

  @pl.loop(0, NCHUNK // 2)
  def _chunk(i):
    k = i * 2
    stage(bufs[1], k + 1)
    finish(bufs[0], k)

    @pl.when(i < NCHUNK // 2 - 1)
    def _():
      stage(bufs[0], k + 2)

    finish(bufs[1], k + 1)


def _chunk_scratch():
  return [
      pltpu.VMEM((P,), jnp.float32),
      pltpu.VMEM((P,), jnp.float32),
      pltpu.VMEM((P,), jnp.float32),
      pltpu.VMEM((6, P), jnp.float32),
      pltpu.VMEM((_NROWS128, 128), jnp.int32),
      pltpu.VMEM((_NIDX, C), jnp.bfloat16),
  ]


_tri_plane_sc = functools.partial(
    pl.kernel,
    out_type=jax.ShapeDtypeStruct((N, 2 * C), jnp.float32),
    mesh=plsc.VectorSubcoreMesh(
        core_axis_name="c", subcore_axis_name="s", num_cores=NC, num_subcores=NS
    ),
    scratch_types=(
        _chunk_scratch() + [pltpu.SemaphoreType.DMA]
        + _chunk_scratch() + [pltpu.SemaphoreType.DMA]
        + [pltpu.VMEM((P, 2 * C), jnp.float32)]
    ),
    compiler_params=pltpu.CompilerParams(
        use_tc_tiling_on_sc=False, needs_layout_passes=False
    ),
)(_body)


def kernel(xyz, planes_xy0, planes_xz0, planes_yz0, planes_xy1, planes_xz1,
           planes_yz1):
  planes = (planes_xy0, planes_xz0, planes_yz0, planes_xy1, planes_xz1,
            planes_yz1)
  perm = [c // 2 + (c % 2) * 16 for c in range(2 * 16)]
  table = jnp.concatenate(
      [p.reshape(C, -1).T[:, perm] for p in planes], axis=0
  ).astype(jnp.bfloat16)
  xs = xyz[:, 0]
  ys = xyz[:, 1]
  zs = xyz[:, 2]
  return _tri_plane_sc(xs, ys, zs, table)
